# baseline (speedup 1.0000x reference)
import functools

import jax
import jax.numpy as jnp
from jax import lax
from jax.experimental import pallas as pl
from jax.experimental.pallas import tpu as pltpu
from jax.experimental.pallas import tpu_sc as plsc

_D = 32
_BATCH = 4096
_HIST = 200
_B = _BATCH * _HIST

_info = plsc.get_sparse_core_info()
_NC, _NS = _info.num_cores, _info.num_subcores
_NW = _NC * _NS
_BPW = _BATCH // _NW
_IPW = _BPW * _HIST
_HBLK = 2
_NBLK = _HIST // _HBLK
_ROWS = _HBLK * _BPW
_PT = _HBLK * 4 * 8 * 128

_mesh = plsc.VectorSubcoreMesh(core_axis_name="c", subcore_axis_name="s")


@functools.partial(
    pl.kernel,
    mesh=_mesh,
    out_type=jax.ShapeDtypeStruct((_HIST * 4 * _NW * 8 * 128,), jnp.float32),
    scratch_types=[
        pltpu.VMEM((_IPW,), jnp.int32),
        [pltpu.VMEM((_ROWS,), jnp.int32) for _ in range(2)],
        [pltpu.VMEM((_ROWS, _D), jnp.float32) for _ in range(2)],
        [pltpu.VMEM((_PT,), jnp.float32) for _ in range(2)],
        [pltpu.SemaphoreType.DMA for _ in range(2)],
        [pltpu.SemaphoreType.DMA for _ in range(2)],
    ],
    compiler_params=pltpu.CompilerParams(
        use_tc_tiling_on_sc=False,
        needs_layout_passes=False,
        disable_bounds_checks=True,
    ),
)
def _embed(idx_hbm, table_hbm, out_hbm, idx_v, hidx, rows, ptile, gsems, ssems):
    wid = lax.axis_index("s") * _NC + lax.axis_index("c")

    pltpu.sync_copy(idx_hbm.at[pl.ds(wid * _IPW, _IPW)], idx_v)

    iota = lax.iota(jnp.int32, 16)
    base200 = [iota * _HIST + 16 * _HIST * k for k in range(8)]
    iota128 = iota * 128

    def build_hidx(i, u):
        h0 = i * _HBLK
        for hh in range(_HBLK):
            for k in range(8):
                v = plsc.load_gather(idx_v, [base200[k] + (h0 + hh)])
                hidx[u][pl.ds(hh * _BPW + k * 16, 16)] = v

    def start_gather(u):
        pltpu.async_copy(table_hbm.at[hidx[u]], rows[u], gsems[u])

    def wait_gather(u):
        pltpu.make_async_copy(table_hbm.at[hidx[u]], rows[u], gsems[u]).wait()

    def store_pairs(i, u):
        res = []
        for hh in range(_HBLK):
            for r in range(4):
                src = ptile[u].at[pl.ds(hh * 4096 + r * 1024, 1024)]
                off = ((((i * _HBLK + hh) * 4 + r) * _NW) + wid) * 1024
                res.append((src, out_hbm.at[pl.ds(off, 1024)]))
        return res

    def start_store(i, u):
        for src, dst in store_pairs(i, u):
            pltpu.async_copy(src, dst, ssems[u])

    def wait_store(i, u):
        for src, dst in store_pairs(i, u):
            pltpu.make_async_copy(src, dst, ssems[u]).wait()

    def transpose(u):
        @plsc.parallel_loop(0, _ROWS * 2, unroll=8)
        def _(g):
            j = g >> 1
            half = g & 1
            x = rows[u][j, pl.ds(half * 16, 16)]
            base = ((j >> 7) << 12) + (half << 11) + (j & 127)
            plsc.store_scatter(ptile[u], [iota128 + base], x)

    build_hidx(0, 0)
    start_gather(0)

    def outer(t, carry):
        for u in range(2):
            i = t * 2 + u

            if u == 0:
                build_hidx(i + 1, 1)
                start_gather(1)
            else:

                @pl.when(t < _NBLK // 2 - 1)
                def _():
                    build_hidx(i + 1, 0)
                    start_gather(0)

            wait_gather(u)

            @pl.when(t >= 1)
            def _():
                wait_store(i - 2, u)

            transpose(u)
            start_store(i, u)
        return carry

    lax.fori_loop(0, _NBLK // 2, outer, 0)

    wait_store(_NBLK - 2, 0)
    wait_store(_NBLK - 1, 1)


def kernel(seq, table):
    flat = seq.reshape(-1)
    p = _embed(flat, table)
    p5 = p.reshape(_HIST, 4, _NW, 8, 128)
    return p5.transpose(2, 4, 0, 1, 3).reshape(_BATCH, _HIST, _D)


# speedup vs baseline: 1.0153x; 1.0153x over previous
import functools

import jax
import jax.numpy as jnp
from jax import lax
from jax.experimental import pallas as pl
from jax.experimental.pallas import tpu as pltpu
from jax.experimental.pallas import tpu_sc as plsc

_D = 32
_BATCH = 4096
_HIST = 200
_B = _BATCH * _HIST

_info = plsc.get_sparse_core_info()
_NC, _NS = _info.num_cores, _info.num_subcores
_NW = _NC * _NS
_BPW = _BATCH // _NW
_IPW = _BPW * _HIST
_HBLK = 2
_NBLK = _HIST // _HBLK
_ROWS = _HBLK * _BPW
_PT = _HBLK * 4 * 8 * 128

_mesh = plsc.VectorSubcoreMesh(core_axis_name="c", subcore_axis_name="s")


@functools.partial(
    pl.kernel,
    mesh=_mesh,
    out_type=jax.ShapeDtypeStruct((_HIST * 4 * _NW * 8 * 128,), jnp.float32),
    scratch_types=[
        pltpu.VMEM((_IPW,), jnp.int32),
        [pltpu.VMEM((_ROWS,), jnp.int32) for _ in range(2)],
        [pltpu.VMEM((_ROWS, _D), jnp.float32) for _ in range(2)],
        [pltpu.VMEM((_PT,), jnp.float32) for _ in range(2)],
        [pltpu.SemaphoreType.DMA for _ in range(2)],
        [pltpu.SemaphoreType.DMA for _ in range(2)],
    ],
    compiler_params=pltpu.CompilerParams(
        use_tc_tiling_on_sc=False,
        needs_layout_passes=False,
        disable_bounds_checks=True,
    ),
)
def _embed(idx_hbm, table_hbm, out_hbm, idx_v, hidx, rows, ptile, gsems, ssems):
    wid = lax.axis_index("s") * _NC + lax.axis_index("c")

    pltpu.sync_copy(idx_hbm.at[pl.ds(wid * _IPW, _IPW)], idx_v)

    iota = lax.iota(jnp.int32, 16)
    base200 = [iota * _HIST + 16 * _HIST * k for k in range(8)]
    iota128 = iota * 128

    def build_hidx(i, u):
        h0 = i * _HBLK
        for hh in range(_HBLK):
            for k in range(8):
                v = plsc.load_gather(idx_v, [base200[k] + (h0 + hh)])
                hidx[u][pl.ds(hh * _BPW + k * 16, 16)] = v

    def start_gather(u):
        pltpu.async_copy(table_hbm.at[hidx[u]], rows[u], gsems[u])

    def wait_gather(u):
        pltpu.make_async_copy(table_hbm.at[hidx[u]], rows[u], gsems[u]).wait()

    def store_pairs(i, u):
        res = []
        for hh in range(_HBLK):
            for r in range(4):
                src = ptile[u].at[pl.ds(hh * 4096 + r * 1024, 1024)]
                off = ((((i * _HBLK + hh) * 4 + r) * _NW) + wid) * 1024
                res.append((src, out_hbm.at[pl.ds(off, 1024)]))
        return res

    def start_store(i, u):
        for src, dst in store_pairs(i, u):
            pltpu.async_copy(src, dst, ssems[u])

    def wait_store(i, u):
        for src, dst in store_pairs(i, u):
            pltpu.make_async_copy(src, dst, ssems[u]).wait()

    def transpose(u):
        for hh in range(_HBLK):
            for half in range(2):
                c_row = hh * _BPW
                c_ds = half * 16
                c_base = hh * 4096 + half * 2048

                @plsc.parallel_loop(0, _BPW, unroll=8)
                def _(l, u=u, c_row=c_row, c_ds=c_ds, c_base=c_base):
                    x = rows[u][c_row + l, pl.ds(c_ds, 16)]
                    plsc.store_scatter(ptile[u], [iota128 + (c_base + l)], x)

    build_hidx(0, 0)
    start_gather(0)

    def outer(t, carry):
        for u in range(2):
            i = t * 2 + u

            if u == 0:
                build_hidx(i + 1, 1)
                start_gather(1)
            else:

                @pl.when(t < _NBLK // 2 - 1)
                def _():
                    build_hidx(i + 1, 0)
                    start_gather(0)

            wait_gather(u)

            @pl.when(t >= 1)
            def _():
                wait_store(i - 2, u)

            transpose(u)
            start_store(i, u)
        return carry

    lax.fori_loop(0, _NBLK // 2, outer, 0)

    wait_store(_NBLK - 2, 0)
    wait_store(_NBLK - 1, 1)


def kernel(seq, table):
    flat = seq.reshape(-1)
    p = _embed(flat, table)
    p5 = p.reshape(_HIST, 4, _NW, 8, 128)
    return p5.transpose(2, 4, 0, 1, 3).reshape(_BATCH, _HIST, _D)


# speedup vs baseline: 1.4455x; 1.4237x over previous
import functools

import jax
import jax.numpy as jnp
from jax import lax
from jax.experimental import pallas as pl
from jax.experimental.pallas import tpu as pltpu
from jax.experimental.pallas import tpu_sc as plsc

_D = 32
_BATCH = 4096
_HIST = 200
_B = _BATCH * _HIST

_info = plsc.get_sparse_core_info()
_NC, _NS = _info.num_cores, _info.num_subcores
_NW = _NC * _NS
_BPW = _BATCH // _NW
_IPW = _BPW * _HIST
_HBLK = 2
_NBLK = _HIST // _HBLK
_ROWS = _HBLK * _BPW
_PT = _HBLK * 4 * 8 * 128

_mesh = plsc.VectorSubcoreMesh(core_axis_name="c", subcore_axis_name="s")


@functools.partial(
    pl.kernel,
    mesh=_mesh,
    out_type=jax.ShapeDtypeStruct((_HIST * 4 * _NW * 8 * 128,), jnp.float32),
    scratch_types=[
        pltpu.VMEM((_IPW,), jnp.int32),
        [pltpu.VMEM((_ROWS,), jnp.int32) for _ in range(2)],
        [pltpu.VMEM((_ROWS, _D), jnp.float32) for _ in range(2)],
        [pltpu.VMEM((_PT,), jnp.float32) for _ in range(2)],
        [pltpu.SemaphoreType.DMA for _ in range(2)],
        [pltpu.SemaphoreType.DMA for _ in range(2)],
    ],
    compiler_params=pltpu.CompilerParams(
        use_tc_tiling_on_sc=False,
        needs_layout_passes=False,
        disable_bounds_checks=True,
    ),
)
def _embed(idx_hbm, table_hbm, out_hbm, idx_v, hidx, rows, ptile, gsems, ssems):
    wid = lax.axis_index("s") * _NC + lax.axis_index("c")

    pltpu.sync_copy(idx_hbm.at[pl.ds(wid * _IPW, _IPW)], idx_v)

    iota = lax.iota(jnp.int32, 16)
    base200 = [iota * _HIST + 16 * _HIST * k for k in range(8)]
    iota128 = iota * 128

    def build_hidx(i, u):
        h0 = i * _HBLK
        for hh in range(_HBLK):
            for k in range(8):
                v = plsc.load_gather(idx_v, [base200[k] + (h0 + hh)])
                hidx[u][pl.ds(hh * _BPW + k * 16, 16)] = v

    def start_gather(u):
        pltpu.async_copy(table_hbm.at[hidx[u]], rows[u], gsems[u])

    def wait_gather(u):
        pltpu.make_async_copy(table_hbm.at[hidx[u]], rows[u], gsems[u]).wait()

    def store_pairs(i, u):
        res = []
        for hh in range(_HBLK):
            for r in range(4):
                src = ptile[u].at[pl.ds(hh * 4096 + r * 1024, 1024)]
                off = ((((i * _HBLK + hh) * 4 + r) * _NW) + wid) * 1024
                res.append((src, out_hbm.at[pl.ds(off, 1024)]))
        return res

    def start_store(i, u):
        for src, dst in store_pairs(i, u):
            pltpu.async_copy(src, dst, ssems[u])

    def wait_store(i, u):
        for src, dst in store_pairs(i, u):
            pltpu.make_async_copy(src, dst, ssems[u]).wait()

    def transpose(u):
        for hh in range(_HBLK):

            @plsc.parallel_loop(0, _D * (_BPW // 16), unroll=8)
            def _(g, u=u, hh=hh):
                d0 = g & 31
                l0 = (g >> 5) << 4
                m = (iota + d0) & 31
                rowv = iota + (hh * _BPW + l0)
                val = plsc.load_gather(rows[u], [rowv, m])
                dst = (m << 7) + rowv + (hh * (4096 - _BPW))
                plsc.store_scatter(ptile[u], [dst], val)

    build_hidx(0, 0)
    start_gather(0)

    def outer(t, carry):
        for u in range(2):
            i = t * 2 + u

            if u == 0:
                build_hidx(i + 1, 1)
                start_gather(1)
            else:

                @pl.when(t < _NBLK // 2 - 1)
                def _():
                    build_hidx(i + 1, 0)
                    start_gather(0)

            wait_gather(u)

            @pl.when(t >= 1)
            def _():
                wait_store(i - 2, u)

            transpose(u)
            start_store(i, u)
        return carry

    lax.fori_loop(0, _NBLK // 2, outer, 0)

    wait_store(_NBLK - 2, 0)
    wait_store(_NBLK - 1, 1)


def kernel(seq, table):
    flat = seq.reshape(-1)
    p = _embed(flat, table)
    p5 = p.reshape(_HIST, 4, _NW, 8, 128)
    return p5.transpose(2, 4, 0, 1, 3).reshape(_BATCH, _HIST, _D)
